# Initial kernel scaffold; baseline (speedup 1.0000x reference)
import jax
import jax.numpy as jnp
from jax.experimental import pallas as pl


def kernel(x, edge_index, W7, b7, W8, b8, W81, b81, W82, b82, W9, b9):
    raise NotImplementedError("write your pallas kernel here")


# speedup vs baseline: 1.1687x; 1.1687x over previous
import functools

import jax
import jax.numpy as jnp
from jax import lax
from jax.experimental import pallas as pl
from jax.experimental.pallas import tpu as pltpu
from jax.experimental.pallas import tpu_sc as plsc

N_NODES = 10000
D_FEAT = 256
N_EDGES = 160000
DT = 128
DP = 32
NC, NS = 2, 16
NW = NC * NS
E_PAD = 163840
CHUNK = 256
EDGES_PER_W = E_PAD // NW
N_CHUNKS = EDGES_PER_W // CHUNK
BE = 2048


def _proj_body(x_ref, w_ref, b_ref, o_ref):
    xw = lax.dot_general(x_ref[...], w_ref[...],
                         (((1,), (0,)), ((), ())),
                         precision=lax.Precision.HIGHEST,
                         preferred_element_type=jnp.float32)
    o_ref[...] = xw + b_ref[...]


def _node_proj(x, ws, bs):
    return pl.pallas_call(
        _proj_body,
        grid=(10,),
        in_specs=[
            pl.BlockSpec((1000, D_FEAT), lambda i: (i, 0)),
            pl.BlockSpec((D_FEAT, DT), lambda i: (0, 0)),
            pl.BlockSpec((1, DT), lambda i: (0, 0)),
        ],
        out_specs=pl.BlockSpec((1000, DT), lambda i: (i, 0)),
        out_shape=jax.ShapeDtypeStruct((N_NODES, DT), jnp.float32),
    )(x, ws, bs)


def _gather_add(table, src, dst):
    mesh = plsc.VectorSubcoreMesh(core_axis_name="c", subcore_axis_name="s")

    @functools.partial(
        pl.kernel, mesh=mesh,
        out_type=jax.ShapeDtypeStruct((E_PAD, DP), jnp.float32),
        scratch_types=[
            pltpu.VMEM((CHUNK,), jnp.int32),
            pltpu.VMEM((CHUNK,), jnp.int32),
            pltpu.VMEM((CHUNK, DT), jnp.float32),
            pltpu.VMEM((CHUNK, DT), jnp.float32),
            pltpu.VMEM((CHUNK, DP), jnp.float32),
            pltpu.SemaphoreType.DMA,
            pltpu.SemaphoreType.DMA,
        ],
    )
    def k(table_hbm, src_hbm, dst_hbm, out_hbm,
          si_v, di_v, rows_s, rows_d, out_v, sem_s, sem_d):
        wid = lax.axis_index("s") * NC + lax.axis_index("c")
        base = wid * EDGES_PER_W

        @pl.loop(0, N_CHUNKS)
        def _(c):
            off = base + c * CHUNK
            pltpu.sync_copy(src_hbm.at[pl.ds(off, CHUNK)], si_v)
            pltpu.sync_copy(dst_hbm.at[pl.ds(off, CHUNK)], di_v)
            cp_s = pltpu.async_copy(table_hbm.at[si_v], rows_s, sem_s)
            cp_d = pltpu.async_copy(table_hbm.at[di_v], rows_d, sem_d)
            cp_s.wait()
            cp_d.wait()

            @pl.loop(0, CHUNK)
            def _(r):
                out_v.at[pl.ds(r, 1), pl.ds(0, 16)][...] = (
                    rows_s.at[pl.ds(r, 1), pl.ds(0, 16)][...]
                    + rows_d.at[pl.ds(r, 1), pl.ds(32, 16)][...])
                out_v.at[pl.ds(r, 1), pl.ds(16, 16)][...] = (
                    rows_s.at[pl.ds(r, 1), pl.ds(16, 16)][...]
                    + rows_d.at[pl.ds(r, 1), pl.ds(48, 16)][...])

            pltpu.sync_copy(out_v, out_hbm.at[pl.ds(off, CHUNK)])

    return k(table, src, dst)


def _leaky(v):
    return jnp.where(v >= 0, v, 0.1 * v)


def _mlp_body(e_ref, w8_ref, b8_ref, w81_ref, b81_ref, w82_ref, b82_ref,
              w9_ref, b9_ref, o_ref):
    dn = (((1,), (0,)), ((), ()))
    h = _leaky(e_ref[...])
    h = _leaky(lax.dot_general(h, w8_ref[...], dn,
                               preferred_element_type=jnp.float32) + b8_ref[...])
    h = _leaky(lax.dot_general(h, w81_ref[...], dn,
                               preferred_element_type=jnp.float32) + b81_ref[...])
    h = _leaky(lax.dot_general(h, w82_ref[...], dn,
                               preferred_element_type=jnp.float32) + b82_ref[...])
    z = lax.dot_general(h, w9_ref[...], dn,
                        preferred_element_type=jnp.float32) + b9_ref[...]
    lane = lax.broadcasted_iota(jnp.int32, z.shape, 1)
    z = jnp.where(lane < 4, z, -1e30)
    m = jnp.max(z, axis=1, keepdims=True)
    ez = jnp.exp(z - m)
    p = ez / jnp.sum(ez, axis=1, keepdims=True)
    o_ref[...] = p[:, :4]


def _mlp(e0, w8p, b8p, w81p, b81p, w82p, b82p, w9p, b9p):
    full = lambda shape: pl.BlockSpec(shape, lambda i: tuple(0 for _ in shape))
    return pl.pallas_call(
        _mlp_body,
        grid=(E_PAD // BE,),
        in_specs=[
            pl.BlockSpec((BE, DP), lambda i: (i, 0)),
            full((DP, 16)), full((1, 16)),
            full((16, 16)), full((1, 16)),
            full((16, 8)), full((1, 8)),
            full((8, 8)), full((1, 8)),
        ],
        out_specs=pl.BlockSpec((BE, 4), lambda i: (i, 0)),
        out_shape=jax.ShapeDtypeStruct((E_PAD, 4), jnp.float32),
    )(e0, w8p, b8p, w81p, b81p, w82p, b82p, w9p, b9p)


def kernel(x, edge_index, W7, b7, W8, b8, W81, b81, W82, b82, W9, b9):
    W7a = W7[:D_FEAT]
    W7b = W7[D_FEAT:]
    ws = jnp.zeros((D_FEAT, DT), jnp.float32)
    ws = ws.at[:, :20].set(W7b)
    ws = ws.at[:, 32:52].set(W7a - W7b)
    bs = jnp.zeros((1, DT), jnp.float32).at[0, 32:52].set(b7)

    table = _node_proj(x, ws, bs)

    pad = ((0, E_PAD - N_EDGES),)
    src = jnp.pad(edge_index[0], pad)
    dst = jnp.pad(edge_index[1], pad)

    e0 = _gather_add(table, src, dst)

    w8p = jnp.zeros((DP, 16), jnp.float32).at[:20, :10].set(W8)
    b8p = jnp.zeros((1, 16), jnp.float32).at[0, :10].set(b8)
    w81p = jnp.zeros((16, 16), jnp.float32).at[:10, :10].set(W81)
    b81p = jnp.zeros((1, 16), jnp.float32).at[0, :10].set(b81)
    w82p = jnp.zeros((16, 8), jnp.float32).at[:10, :5].set(W82)
    b82p = jnp.zeros((1, 8), jnp.float32).at[0, :5].set(b82)
    w9p = jnp.zeros((8, 8), jnp.float32).at[:5, :4].set(W9)
    b9p = jnp.zeros((1, 8), jnp.float32).at[0, :4].set(b9)

    out = _mlp(e0, w8p, b8p, w81p, b81p, w82p, b82p, w9p, b9p)
    return out[:N_EDGES]


# speedup vs baseline: 1.2678x; 1.0848x over previous
import functools

import jax
import jax.numpy as jnp
from jax import lax
from jax.experimental import pallas as pl
from jax.experimental.pallas import tpu as pltpu
from jax.experimental.pallas import tpu_sc as plsc

N_NODES = 10000
D_FEAT = 256
N_EDGES = 160000
DT = 128
DP = 32
NC, NS = 2, 16
NW = NC * NS
E_PAD = 163840
CHUNK = 128
EDGES_PER_W = E_PAD // NW
N_CHUNKS = EDGES_PER_W // CHUNK
BE = 2048


def _proj_body(x_ref, w_ref, b_ref, o_ref):
    xw = lax.dot_general(x_ref[...], w_ref[...],
                         (((1,), (0,)), ((), ())),
                         precision=lax.Precision.HIGHEST,
                         preferred_element_type=jnp.float32)
    o_ref[...] = xw + b_ref[...]


def _node_proj(x, ws, bs):
    return pl.pallas_call(
        _proj_body,
        grid=(10,),
        in_specs=[
            pl.BlockSpec((1000, D_FEAT), lambda i: (i, 0)),
            pl.BlockSpec((D_FEAT, DT), lambda i: (0, 0)),
            pl.BlockSpec((1, DT), lambda i: (0, 0)),
        ],
        out_specs=pl.BlockSpec((1000, DT), lambda i: (i, 0)),
        out_shape=jax.ShapeDtypeStruct((N_NODES, DT), jnp.float32),
    )(x, ws, bs)


def _gather_add(table, src, dst):
    mesh = plsc.VectorSubcoreMesh(core_axis_name="c", subcore_axis_name="s")

    @functools.partial(
        pl.kernel, mesh=mesh,
        out_type=jax.ShapeDtypeStruct((E_PAD, DP), jnp.float32),
        scratch_types=[
            pltpu.VMEM((EDGES_PER_W,), jnp.int32),
            pltpu.VMEM((EDGES_PER_W,), jnp.int32),
            pltpu.VMEM((2, CHUNK, DT), jnp.float32),
            pltpu.VMEM((2, CHUNK, DT), jnp.float32),
            pltpu.VMEM((2, CHUNK, DP), jnp.float32),
            pltpu.SemaphoreType.DMA,
            pltpu.SemaphoreType.DMA,
            pltpu.SemaphoreType.DMA,
            pltpu.SemaphoreType.DMA,
            pltpu.SemaphoreType.DMA,
        ],
    )
    def k(table_hbm, src_hbm, dst_hbm, out_hbm,
          si_v, di_v, rows_s, rows_d, out_v, sem_i, sem_g0, sem_g1,
          sem_w0, sem_w1):
        wid = lax.axis_index("s") * NC + lax.axis_index("c")
        base = wid * EDGES_PER_W
        sem_g = (sem_g0, sem_g1)
        sem_w = (sem_w0, sem_w1)

        cp_si = pltpu.async_copy(src_hbm.at[pl.ds(base, EDGES_PER_W)], si_v, sem_i)
        cp_di = pltpu.async_copy(dst_hbm.at[pl.ds(base, EDGES_PER_W)], di_v, sem_i)
        cp_si.wait()
        cp_di.wait()

        def issue(c, b):
            isl = pl.ds(c * CHUNK, CHUNK)
            pltpu.async_copy(table_hbm.at[si_v.at[isl]], rows_s.at[b], sem_g[b])
            pltpu.async_copy(table_hbm.at[di_v.at[isl]], rows_d.at[b], sem_g[b])

        def wait_gathers(b):
            pltpu.make_async_copy(table_hbm.at[si_v.at[pl.ds(0, CHUNK)]],
                                  rows_s.at[b], sem_g[b]).wait()
            pltpu.make_async_copy(table_hbm.at[di_v.at[pl.ds(0, CHUNK)]],
                                  rows_d.at[b], sem_g[b]).wait()

        def add_and_write(c, b):
            @pl.loop(0, CHUNK, step=8)
            def _(r0):
                for u in range(8):
                    r = r0 + u
                    for j in (0, 16):
                        out_v.at[b, pl.ds(r, 1), pl.ds(j, 16)][...] = (
                            rows_s.at[b, pl.ds(r, 1), pl.ds(j, 16)][...]
                            + rows_d.at[b, pl.ds(r, 1), pl.ds(32 + j, 16)][...])

            @pl.when(c >= 2)
            def _():
                pltpu.make_async_copy(
                    out_v.at[b], out_hbm.at[pl.ds(base, CHUNK)], sem_w[b]).wait()
            pltpu.async_copy(out_v.at[b],
                             out_hbm.at[pl.ds(base + c * CHUNK, CHUNK)], sem_w[b])

        issue(0, 0)

        @pl.loop(0, N_CHUNKS, step=2)
        def _(c):
            issue(c + 1, 1)
            wait_gathers(0)
            add_and_write(c, 0)

            @pl.when(c + 2 < N_CHUNKS)
            def _():
                issue(c + 2, 0)
            wait_gathers(1)
            add_and_write(c + 1, 1)

        pltpu.make_async_copy(out_v.at[0], out_hbm.at[pl.ds(base, CHUNK)],
                              sem_w[0]).wait()
        pltpu.make_async_copy(out_v.at[1], out_hbm.at[pl.ds(base, CHUNK)],
                              sem_w[1]).wait()

    return k(table, src, dst)


def _leaky(v):
    return jnp.where(v >= 0, v, 0.1 * v)


def _mlp_body(e_ref, w8_ref, b8_ref, w81_ref, b81_ref, w82_ref, b82_ref,
              w9_ref, b9_ref, o_ref):
    dn = (((1,), (0,)), ((), ()))
    h = _leaky(e_ref[...])
    h = _leaky(lax.dot_general(h, w8_ref[...], dn,
                               preferred_element_type=jnp.float32) + b8_ref[...])
    h = _leaky(lax.dot_general(h, w81_ref[...], dn,
                               preferred_element_type=jnp.float32) + b81_ref[...])
    h = _leaky(lax.dot_general(h, w82_ref[...], dn,
                               preferred_element_type=jnp.float32) + b82_ref[...])
    z = lax.dot_general(h, w9_ref[...], dn,
                        preferred_element_type=jnp.float32) + b9_ref[...]
    lane = lax.broadcasted_iota(jnp.int32, z.shape, 1)
    z = jnp.where(lane < 4, z, -1e30)
    m = jnp.max(z, axis=1, keepdims=True)
    ez = jnp.exp(z - m)
    p = ez / jnp.sum(ez, axis=1, keepdims=True)
    o_ref[...] = p[:, :4]


def _mlp(e0, w8p, b8p, w81p, b81p, w82p, b82p, w9p, b9p):
    full = lambda shape: pl.BlockSpec(shape, lambda i: tuple(0 for _ in shape))
    return pl.pallas_call(
        _mlp_body,
        grid=(E_PAD // BE,),
        in_specs=[
            pl.BlockSpec((BE, DP), lambda i: (i, 0)),
            full((DP, 16)), full((1, 16)),
            full((16, 16)), full((1, 16)),
            full((16, 8)), full((1, 8)),
            full((8, 8)), full((1, 8)),
        ],
        out_specs=pl.BlockSpec((BE, 4), lambda i: (i, 0)),
        out_shape=jax.ShapeDtypeStruct((E_PAD, 4), jnp.float32),
    )(e0, w8p, b8p, w81p, b81p, w82p, b82p, w9p, b9p)


def kernel(x, edge_index, W7, b7, W8, b8, W81, b81, W82, b82, W9, b9):
    W7a = W7[:D_FEAT]
    W7b = W7[D_FEAT:]
    ws = jnp.zeros((D_FEAT, DT), jnp.float32)
    ws = ws.at[:, :20].set(W7b)
    ws = ws.at[:, 32:52].set(W7a - W7b)
    bs = jnp.zeros((1, DT), jnp.float32).at[0, 32:52].set(b7)

    table = _node_proj(x, ws, bs)

    pad = ((0, E_PAD - N_EDGES),)
    src = jnp.pad(edge_index[0], pad)
    dst = jnp.pad(edge_index[1], pad)

    e0 = _gather_add(table, src, dst)

    w8p = jnp.zeros((DP, 16), jnp.float32).at[:20, :10].set(W8)
    b8p = jnp.zeros((1, 16), jnp.float32).at[0, :10].set(b8)
    w81p = jnp.zeros((16, 16), jnp.float32).at[:10, :10].set(W81)
    b81p = jnp.zeros((1, 16), jnp.float32).at[0, :10].set(b81)
    w82p = jnp.zeros((16, 8), jnp.float32).at[:10, :5].set(W82)
    b82p = jnp.zeros((1, 8), jnp.float32).at[0, :5].set(b82)
    w9p = jnp.zeros((8, 8), jnp.float32).at[:5, :4].set(W9)
    b9p = jnp.zeros((1, 8), jnp.float32).at[0, :4].set(b9)

    out = _mlp(e0, w8p, b8p, w81p, b81p, w82p, b82p, w9p, b9p)
    return out[:N_EDGES]


# speedup vs baseline: 1.6170x; 1.2754x over previous
import functools

import jax
import jax.numpy as jnp
from jax import lax
from jax.experimental import pallas as pl
from jax.experimental.pallas import tpu as pltpu
from jax.experimental.pallas import tpu_sc as plsc

N_NODES = 10000
D_FEAT = 256
N_EDGES = 160000
DT = 128
DP = 32
NC, NS = 2, 16
NW = NC * NS
E_PAD = 163840
CHUNK = 128
N0, N1 = 68, 12
NMAX = max(N0, N1)
IDX_PRELOAD = NMAX * CHUNK
BE = 4000


def _proj_body(x_ref, w_ref, b_ref, o_ref):
    xw = lax.dot_general(x_ref[...], w_ref[...],
                         (((1,), (0,)), ((), ())),
                         precision=lax.Precision.HIGHEST,
                         preferred_element_type=jnp.float32)
    o_ref[...] = xw + b_ref[...]


def _node_proj(x, ws, bs):
    return pl.pallas_call(
        _proj_body,
        grid=(10,),
        in_specs=[
            pl.BlockSpec((1000, D_FEAT), lambda i: (i, 0)),
            pl.BlockSpec((D_FEAT, DT), lambda i: (0, 0)),
            pl.BlockSpec((1, DT), lambda i: (0, 0)),
        ],
        out_specs=pl.BlockSpec((1000, DT), lambda i: (i, 0)),
        out_shape=jax.ShapeDtypeStruct((N_NODES, DT), jnp.float32),
    )(x, ws, bs)


def _gather_add(table, src, dst):
    mesh = plsc.VectorSubcoreMesh(core_axis_name="c", subcore_axis_name="s")

    @functools.partial(
        pl.kernel, mesh=mesh,
        out_type=jax.ShapeDtypeStruct((E_PAD, DP), jnp.float32),
        scratch_types=[
            pltpu.VMEM((IDX_PRELOAD,), jnp.int32),
            pltpu.VMEM((IDX_PRELOAD,), jnp.int32),
            pltpu.VMEM((2, CHUNK, DT), jnp.float32),
            pltpu.VMEM((2, CHUNK, DT), jnp.float32),
            pltpu.VMEM((2, CHUNK, DP), jnp.float32),
            pltpu.SemaphoreType.DMA,
            pltpu.SemaphoreType.DMA,
            pltpu.SemaphoreType.DMA,
            pltpu.SemaphoreType.DMA,
            pltpu.SemaphoreType.DMA,
        ],
    )
    def k(table_hbm, src_hbm, dst_hbm, out_hbm,
          si_v, di_v, rows_s, rows_d, out_v, sem_i, sem_g0, sem_g1,
          sem_w0, sem_w1):
        ci = lax.axis_index("c")
        s = lax.axis_index("s")
        my_n = jnp.where(ci == 0, N0, N1)
        base_chunk = jnp.where(ci == 0, s * N0, 16 * N0 + s * N1)
        base = base_chunk * CHUNK
        sem_g = (sem_g0, sem_g1)
        sem_w = (sem_w0, sem_w1)

        cp_si = pltpu.async_copy(src_hbm.at[pl.ds(base, IDX_PRELOAD)], si_v, sem_i)
        cp_di = pltpu.async_copy(dst_hbm.at[pl.ds(base, IDX_PRELOAD)], di_v, sem_i)
        cp_si.wait()
        cp_di.wait()

        def issue(c, b):
            isl = pl.ds(c * CHUNK, CHUNK)
            pltpu.async_copy(table_hbm.at[si_v.at[isl]], rows_s.at[b], sem_g[b])
            pltpu.async_copy(table_hbm.at[di_v.at[isl]], rows_d.at[b], sem_g[b])

        def wait_gathers(b):
            pltpu.make_async_copy(table_hbm.at[si_v.at[pl.ds(0, CHUNK)]],
                                  rows_s.at[b], sem_g[b]).wait()
            pltpu.make_async_copy(table_hbm.at[di_v.at[pl.ds(0, CHUNK)]],
                                  rows_d.at[b], sem_g[b]).wait()

        def add_and_write(c, b):
            @pl.loop(0, CHUNK, step=8)
            def _(r0):
                for u in range(8):
                    r = r0 + u
                    for j in (0, 16):
                        out_v.at[b, pl.ds(r, 1), pl.ds(j, 16)][...] = (
                            rows_s.at[b, pl.ds(r, 1), pl.ds(j, 16)][...]
                            + rows_d.at[b, pl.ds(r, 1), pl.ds(32 + j, 16)][...])

            @pl.when(c >= 2)
            def _():
                pltpu.make_async_copy(
                    out_v.at[b], out_hbm.at[pl.ds(base, CHUNK)], sem_w[b]).wait()
            pltpu.async_copy(out_v.at[b],
                             out_hbm.at[pl.ds(base + c * CHUNK, CHUNK)], sem_w[b])

        issue(0, 0)

        @pl.loop(0, my_n, step=2)
        def _(c):
            issue(c + 1, 1)
            wait_gathers(0)
            add_and_write(c, 0)

            @pl.when(c + 2 < my_n)
            def _():
                issue(c + 2, 0)
            wait_gathers(1)
            add_and_write(c + 1, 1)

        pltpu.make_async_copy(out_v.at[0], out_hbm.at[pl.ds(base, CHUNK)],
                              sem_w[0]).wait()
        pltpu.make_async_copy(out_v.at[1], out_hbm.at[pl.ds(base, CHUNK)],
                              sem_w[1]).wait()

    return k(table, src, dst)


def _leaky(v):
    return jnp.where(v >= 0, v, 0.1 * v)


def _mlp_body(e_ref, w8_ref, b8_ref, w81_ref, b81_ref, w82_ref, b82_ref,
              w9_ref, b9_ref, o_ref):
    dn = (((1,), (0,)), ((), ()))
    h = _leaky(e_ref[...])
    h = _leaky(lax.dot_general(h, w8_ref[...], dn,
                               preferred_element_type=jnp.float32) + b8_ref[...])
    h = _leaky(lax.dot_general(h, w81_ref[...], dn,
                               preferred_element_type=jnp.float32) + b81_ref[...])
    h = _leaky(lax.dot_general(h, w82_ref[...], dn,
                               preferred_element_type=jnp.float32) + b82_ref[...])
    z = lax.dot_general(h, w9_ref[...], dn,
                        preferred_element_type=jnp.float32) + b9_ref[...]
    lane = lax.broadcasted_iota(jnp.int32, z.shape, 1)
    z = jnp.where(lane < 4, z, -1e30)
    m = jnp.max(z, axis=1, keepdims=True)
    ez = jnp.exp(z - m)
    p = ez / jnp.sum(ez, axis=1, keepdims=True)
    o_ref[...] = p[:, :4]


def _mlp(e0, w8p, b8p, w81p, b81p, w82p, b82p, w9p, b9p):
    full = lambda shape: pl.BlockSpec(shape, lambda i: tuple(0 for _ in shape))
    return pl.pallas_call(
        _mlp_body,
        grid=(N_EDGES // BE,),
        in_specs=[
            pl.BlockSpec((BE, DP), lambda i: (i, 0)),
            full((DP, 16)), full((1, 16)),
            full((16, 16)), full((1, 16)),
            full((16, 8)), full((1, 8)),
            full((8, 8)), full((1, 8)),
        ],
        out_specs=pl.BlockSpec((BE, 4), lambda i: (i, 0)),
        out_shape=jax.ShapeDtypeStruct((N_EDGES, 4), jnp.float32),
    )(e0, w8p, b8p, w81p, b81p, w82p, b82p, w9p, b9p)


def kernel(x, edge_index, W7, b7, W8, b8, W81, b81, W82, b82, W9, b9):
    W7a = W7[:D_FEAT]
    W7b = W7[D_FEAT:]
    ws = jnp.zeros((D_FEAT, DT), jnp.float32)
    ws = ws.at[:, :20].set(W7b)
    ws = ws.at[:, 32:52].set(W7a - W7b)
    bs = jnp.zeros((1, DT), jnp.float32).at[0, 32:52].set(b7)

    table = _node_proj(x, ws, bs)

    pad = ((0, E_PAD + IDX_PRELOAD - N_EDGES),)
    src = jnp.pad(edge_index[0], pad)
    dst = jnp.pad(edge_index[1], pad)

    e0 = _gather_add(table, src, dst)

    w8p = jnp.zeros((DP, 16), jnp.float32).at[:20, :10].set(W8)
    b8p = jnp.zeros((1, 16), jnp.float32).at[0, :10].set(b8)
    w81p = jnp.zeros((16, 16), jnp.float32).at[:10, :10].set(W81)
    b81p = jnp.zeros((1, 16), jnp.float32).at[0, :10].set(b81)
    w82p = jnp.zeros((16, 8), jnp.float32).at[:10, :5].set(W82)
    b82p = jnp.zeros((1, 8), jnp.float32).at[0, :5].set(b82)
    w9p = jnp.zeros((8, 8), jnp.float32).at[:5, :4].set(W9)
    b9p = jnp.zeros((1, 8), jnp.float32).at[0, :4].set(b9)

    return _mlp(e0, w8p, b8p, w81p, b81p, w82p, b82p, w9p, b9p)
